# Initial kernel scaffold; baseline (speedup 1.0000x reference)
import jax
import jax.numpy as jnp
from jax.experimental import pallas as pl


def kernel(atomic_embedding, pair_indices, f_ij, f_ij_cutoff, W_in, Wf1, bf1, Wf2, bf2, W2, b2, W3, b3):
    raise NotImplementedError("write your pallas kernel here")


# speedup vs baseline: 2.3433x; 2.3433x over previous
import functools

import jax
import jax.numpy as jnp
from jax import lax
from jax.experimental import pallas as pl
from jax.experimental.pallas import tpu as pltpu
from jax.experimental.pallas import tpu_sc as plsc

N = 10000
E = 320000
F = 128
RBF = 20

NC = 2
NS = 16
NW = NC * NS
E_PER_W = E // NW
B = 80
NBLK = E_PER_W // B
N_PAD = 10240
ROWS_PER_TILE = N_PAD // NS

_LOG2 = 0.6931471805599453


def _ssp(x):
    return jnp.logaddexp(x, 0.0) - _LOG2


def _emb_body(x_ref, w_ref, o_ref):
    o_ref[...] = jnp.dot(x_ref[...], w_ref[...],
                         preferred_element_type=jnp.float32)


def _filter_body(f_ref, c_ref, wf1_ref, bf1_ref, wf2_ref, bf2_ref, o_ref):
    h = _ssp(jnp.dot(f_ref[...], wf1_ref[...],
                     preferred_element_type=jnp.float32) + bf1_ref[...])
    w = jnp.dot(h, wf2_ref[...],
                preferred_element_type=jnp.float32) + bf2_ref[...]
    o_ref[...] = w * c_ref[...]


def _out_body(p_ref, w2_ref, b2_ref, w3_ref, b3_ref, o_ref):
    agg = p_ref[0, :N, :] + p_ref[1, :N, :]
    h = _ssp(jnp.dot(agg, w2_ref[...],
                     preferred_element_type=jnp.float32) + b2_ref[...])
    o_ref[...] = jnp.dot(h, w3_ref[...],
                         preferred_element_type=jnp.float32) + b3_ref[...]


_BE = 8000


def _filter_net(f_ij, f_ij_cutoff, Wf1, bf1, Wf2, bf2):
    grid = (E // _BE,)
    return pl.pallas_call(
        _filter_body,
        grid=grid,
        in_specs=[
            pl.BlockSpec((_BE, RBF), lambda i: (i, 0)),
            pl.BlockSpec((_BE, 1), lambda i: (i, 0)),
            pl.BlockSpec((RBF, F), lambda i: (0, 0)),
            pl.BlockSpec((1, F), lambda i: (0, 0)),
            pl.BlockSpec((F, F), lambda i: (0, 0)),
            pl.BlockSpec((1, F), lambda i: (0, 0)),
        ],
        out_specs=pl.BlockSpec((_BE, F), lambda i: (i, 0)),
        out_shape=jax.ShapeDtypeStruct((E, F), jnp.float32),
    )(f_ij, f_ij_cutoff, Wf1, bf1, Wf2, bf2)


_sc_mesh = plsc.VectorSubcoreMesh(core_axis_name="c", subcore_axis_name="s")


@functools.partial(
    pl.kernel,
    out_type=jax.ShapeDtypeStruct((NC, N_PAD, F), jnp.float32),
    mesh=_sc_mesh,
    scratch_types=[
        pltpu.VMEM((B,), jnp.int32),
        pltpu.VMEM((B,), jnp.int32),
        pltpu.VMEM((B, F), jnp.float32),
        pltpu.VMEM((B, F), jnp.float32),
        pltpu.VMEM_SHARED((N_PAD, F), jnp.float32),
    ],
)
def _sc_agg(emb_hbm, idxj_hbm, idxi_hbm, w_hbm, zeros_hbm, out_hbm,
            idxj_v, idxi_v, rows_v, w_v, acc_sh):
    cid = lax.axis_index("c")
    sid = lax.axis_index("s")
    wid = sid * NC + cid
    r0 = sid * ROWS_PER_TILE

    pltpu.sync_copy(zeros_hbm.at[pl.ds(r0, ROWS_PER_TILE)],
                    acc_sh.at[pl.ds(r0, ROWS_PER_TILE)])
    plsc.subcore_barrier()

    base0 = wid * E_PER_W

    @pl.loop(0, NBLK)
    def _blk(blk):
        base = base0 + blk * B
        pltpu.sync_copy(idxj_hbm.at[pl.ds(base, B)], idxj_v)
        pltpu.sync_copy(idxi_hbm.at[pl.ds(base, B)], idxi_v)
        pltpu.sync_copy(emb_hbm.at[idxj_v], rows_v)
        pltpu.sync_copy(w_hbm.at[pl.ds(base, B)], w_v)

        @pl.loop(0, B)
        def _p(p):
            for c in range(0, F, 16):
                slc = (pl.ds(p, 1), pl.ds(c, 16))
                rows_v.at[slc][...] = rows_v.at[slc][...] * w_v.at[slc][...]

        pltpu.sync_copy(rows_v, acc_sh.at[idxi_v], add=True)

    plsc.subcore_barrier()
    pltpu.sync_copy(acc_sh.at[pl.ds(r0, ROWS_PER_TILE)],
                    out_hbm.at[cid, pl.ds(r0, ROWS_PER_TILE)])


def kernel(atomic_embedding, pair_indices, f_ij, f_ij_cutoff,
           W_in, Wf1, bf1, Wf2, bf2, W2, b2, W3, b3):
    emb = pl.pallas_call(
        _emb_body,
        out_shape=jax.ShapeDtypeStruct((N, F), jnp.float32),
    )(atomic_embedding, W_in)

    w_ij = _filter_net(f_ij, f_ij_cutoff, Wf1, bf1.reshape(1, F),
                       Wf2, bf2.reshape(1, F))

    idx_i = pair_indices[0]
    idx_j = pair_indices[1]
    zeros = jnp.zeros((N_PAD, F), jnp.float32)

    partials = _sc_agg(emb, idx_j, idx_i, w_ij, zeros)

    out = pl.pallas_call(
        _out_body,
        out_shape=jax.ShapeDtypeStruct((N, F), jnp.float32),
    )(partials, W2, b2.reshape(1, F), W3, b3.reshape(1, F))
    return out


# speedup vs baseline: 2.7810x; 1.1868x over previous
import functools

import jax
import jax.numpy as jnp
from jax import lax
from jax.experimental import pallas as pl
from jax.experimental.pallas import tpu as pltpu
from jax.experimental.pallas import tpu_sc as plsc

N = 10000
E = 320000
F = 128
RBF = 20

NC = 2
NS = 16
NW = NC * NS
E_PER_W = E // NW
B = 40
NBLK = E_PER_W // B
N_PAD = 10240
ROWS_PER_TILE = N_PAD // NS

NBUF = 3
IDEPTH = 6

_LOG2 = 0.6931471805599453


def _ssp(x):
    return jnp.logaddexp(x, 0.0) - _LOG2


def _emb_body(x_ref, w_ref, o_ref):
    o_ref[...] = jnp.dot(x_ref[...], w_ref[...],
                         preferred_element_type=jnp.float32)


def _filter_body(f_ref, c_ref, wf1_ref, bf1_ref, wf2_ref, bf2_ref, o_ref):
    h = _ssp(jnp.dot(f_ref[...], wf1_ref[...],
                     preferred_element_type=jnp.float32) + bf1_ref[...])
    w = jnp.dot(h, wf2_ref[...],
                preferred_element_type=jnp.float32) + bf2_ref[...]
    o_ref[...] = w * c_ref[...]


def _out_body(p_ref, w2_ref, b2_ref, w3_ref, b3_ref, o_ref):
    agg = p_ref[0, :N, :] + p_ref[1, :N, :]
    h = _ssp(jnp.dot(agg, w2_ref[...],
                     preferred_element_type=jnp.float32) + b2_ref[...])
    o_ref[...] = jnp.dot(h, w3_ref[...],
                         preferred_element_type=jnp.float32) + b3_ref[...]


_BE = 8000


def _filter_net(f_ij, f_ij_cutoff, Wf1, bf1, Wf2, bf2):
    grid = (E // _BE,)
    return pl.pallas_call(
        _filter_body,
        grid=grid,
        in_specs=[
            pl.BlockSpec((_BE, RBF), lambda i: (i, 0)),
            pl.BlockSpec((_BE, 1), lambda i: (i, 0)),
            pl.BlockSpec((RBF, F), lambda i: (0, 0)),
            pl.BlockSpec((1, F), lambda i: (0, 0)),
            pl.BlockSpec((F, F), lambda i: (0, 0)),
            pl.BlockSpec((1, F), lambda i: (0, 0)),
        ],
        out_specs=pl.BlockSpec((_BE, F), lambda i: (i, 0)),
        out_shape=jax.ShapeDtypeStruct((E, F), jnp.float32),
    )(f_ij, f_ij_cutoff, Wf1, bf1, Wf2, bf2)


_sc_mesh = plsc.VectorSubcoreMesh(core_axis_name="c", subcore_axis_name="s")


@functools.partial(
    pl.kernel,
    out_type=jax.ShapeDtypeStruct((NC, N_PAD, F), jnp.float32),
    mesh=_sc_mesh,
    scratch_types=[
        [pltpu.VMEM((B,), jnp.int32)] * IDEPTH,
        [pltpu.VMEM((B,), jnp.int32)] * IDEPTH,
        [pltpu.VMEM((B, F), jnp.float32)] * NBUF,
        [pltpu.VMEM((B, F), jnp.float32)] * NBUF,
        pltpu.VMEM_SHARED((N_PAD, F), jnp.float32),
        pltpu.SemaphoreType.DMA((IDEPTH,)),
        pltpu.SemaphoreType.DMA((IDEPTH,)),
        pltpu.SemaphoreType.DMA((NBUF,)),
        pltpu.SemaphoreType.DMA((NBUF,)),
        pltpu.SemaphoreType.DMA((NBUF,)),
    ],
)
def _sc_agg(emb_hbm, idxj_hbm, idxi_hbm, w_hbm, zeros_hbm, out_hbm,
            idxj_r, idxi_r, rows, wv, acc_sh,
            ij_sem, ii_sem, g_sem, w_sem, s_sem):
    cid = lax.axis_index("c")
    sid = lax.axis_index("s")
    wid = sid * NC + cid
    r0 = sid * ROWS_PER_TILE
    base0 = wid * E_PER_W

    def start_idx(kk, i6):
        pltpu.async_copy(idxj_hbm.at[pl.ds(base0 + kk * B, B)],
                         idxj_r[i6], ij_sem.at[i6])
        pltpu.async_copy(idxi_hbm.at[pl.ds(base0 + kk * B, B)],
                         idxi_r[i6], ii_sem.at[i6])

    def wait_idxj(i6):
        pltpu.make_async_copy(idxj_hbm.at[pl.ds(0, B)],
                              idxj_r[i6], ij_sem.at[i6]).wait()

    def wait_idxi(i6):
        pltpu.make_async_copy(idxi_hbm.at[pl.ds(0, B)],
                              idxi_r[i6], ii_sem.at[i6]).wait()

    def start_data(kk, i3, i6):
        pltpu.async_copy(emb_hbm.at[idxj_r[i6]], rows[i3], g_sem.at[i3])
        pltpu.async_copy(w_hbm.at[pl.ds(base0 + kk * B, B)], wv[i3],
                         w_sem.at[i3])

    def wait_scatter(i3):
        pltpu.make_async_copy(rows[i3], acc_sh.at[idxi_r[0]],
                              s_sem.at[i3]).wait()

    def process(kq, q, guard):
        i3, i6 = q % NBUF, q % IDEPTH
        n3, n6 = (q + 1) % NBUF, (q + 1) % IDEPTH

        if guard is None or kq + 2 < NBLK:
            start_idx(kq + 2, (q + 2) % IDEPTH)
        if guard is None or kq + 1 < NBLK:
            if guard is None:
                @pl.when(kq >= NBUF - 1)
                def _():
                    wait_scatter(n3)
            elif kq + 1 >= NBUF:
                wait_scatter(n3)
            wait_idxj(n6)
            start_data(kq + 1, n3, n6)

        pltpu.make_async_copy(emb_hbm.at[idxj_r[i6]], rows[i3],
                              g_sem.at[i3]).wait()
        pltpu.make_async_copy(w_hbm.at[pl.ds(0, B)], wv[i3],
                              w_sem.at[i3]).wait()

        @pl.loop(0, B, unroll=2)
        def _p(p):
            for c in range(0, F, 16):
                slc = (pl.ds(p, 1), pl.ds(c, 16))
                rows[i3].at[slc][...] = (rows[i3].at[slc][...]
                                         * wv[i3].at[slc][...])

        wait_idxi(i6)
        pltpu.async_copy(rows[i3], acc_sh.at[idxi_r[i6]],
                         s_sem.at[i3], add=True)

    start_idx(0, 0)
    start_idx(1, 1)
    wait_idxj(0)
    start_data(0, 0, 0)
    pltpu.sync_copy(zeros_hbm.at[pl.ds(r0, ROWS_PER_TILE)],
                    acc_sh.at[pl.ds(r0, ROWS_PER_TILE)])
    plsc.subcore_barrier()

    @pl.loop(0, NBLK - 4, step=IDEPTH)
    def _blk(kk):
        for q in range(IDEPTH):
            process(kk + q, q, None)

    for kq in range(NBLK - 4, NBLK):
        process(kq, kq % IDEPTH, "tail")

    for kq in range(NBLK - NBUF, NBLK):
        wait_scatter(kq % NBUF)

    plsc.subcore_barrier()
    pltpu.sync_copy(acc_sh.at[pl.ds(r0, ROWS_PER_TILE)],
                    out_hbm.at[cid, pl.ds(r0, ROWS_PER_TILE)])


def kernel(atomic_embedding, pair_indices, f_ij, f_ij_cutoff,
           W_in, Wf1, bf1, Wf2, bf2, W2, b2, W3, b3):
    emb = pl.pallas_call(
        _emb_body,
        out_shape=jax.ShapeDtypeStruct((N, F), jnp.float32),
    )(atomic_embedding, W_in)

    w_ij = _filter_net(f_ij, f_ij_cutoff, Wf1, bf1.reshape(1, F),
                       Wf2, bf2.reshape(1, F))

    idx_i = pair_indices[0]
    idx_j = pair_indices[1]
    zeros = jnp.zeros((N_PAD, F), jnp.float32)

    partials = _sc_agg(emb, idx_j, idx_i, w_ij, zeros)

    out = pl.pallas_call(
        _out_body,
        out_shape=jax.ShapeDtypeStruct((N, F), jnp.float32),
    )(partials, W2, b2.reshape(1, F), W3, b3.reshape(1, F))
    return out


# speedup vs baseline: 3.7016x; 1.3310x over previous
import functools

import jax
import jax.numpy as jnp
from jax import lax
from jax.experimental import pallas as pl
from jax.experimental.pallas import tpu as pltpu
from jax.experimental.pallas import tpu_sc as plsc

N = 10000
E = 320000
F = 128
RBF = 20

NC = 2
NS = 16
NW = NC * NS
E_PER_W = E // NW
B = 40
NBLK = E_PER_W // B
N_PAD = 10240
ROWS_PER_TILE = N_PAD // NS

NBUF = 3
IDEPTH = 6

_LOG2 = 0.6931471805599453


def _ssp(x):
    return jnp.logaddexp(x, 0.0) - _LOG2


def _emb_body(x_ref, w_ref, o_ref):
    o_ref[...] = jnp.dot(x_ref[...], w_ref[...],
                         preferred_element_type=jnp.float32)


def _filter_body(f_ref, c_ref, wf1_ref, bf1_ref, wf2_ref, bf2_ref, o_ref):
    h = _ssp(jnp.dot(f_ref[...], wf1_ref[...],
                     preferred_element_type=jnp.float32) + bf1_ref[...])
    w = jnp.dot(h, wf2_ref[...],
                preferred_element_type=jnp.float32) + bf2_ref[...]
    o_ref[...] = w * c_ref[...]


def _out_body(p_ref, w2_ref, b2_ref, w3_ref, b3_ref, o_ref):
    agg = p_ref[0, :N, :] + p_ref[1, :N, :]
    h = _ssp(jnp.dot(agg, w2_ref[...],
                     preferred_element_type=jnp.float32) + b2_ref[...])
    o_ref[...] = jnp.dot(h, w3_ref[...],
                         preferred_element_type=jnp.float32) + b3_ref[...]


_BE = 8000


def _filter_net(f_ij, f_ij_cutoff, Wf1, bf1, Wf2, bf2):
    grid = (E // _BE,)
    return pl.pallas_call(
        _filter_body,
        grid=grid,
        in_specs=[
            pl.BlockSpec((_BE, RBF), lambda i: (i, 0)),
            pl.BlockSpec((_BE, 1), lambda i: (i, 0)),
            pl.BlockSpec((RBF, F), lambda i: (0, 0)),
            pl.BlockSpec((1, F), lambda i: (0, 0)),
            pl.BlockSpec((F, F), lambda i: (0, 0)),
            pl.BlockSpec((1, F), lambda i: (0, 0)),
        ],
        out_specs=pl.BlockSpec((_BE, F), lambda i: (i, 0)),
        out_shape=jax.ShapeDtypeStruct((E, F), jnp.float32),
    )(f_ij, f_ij_cutoff, Wf1, bf1, Wf2, bf2)


_sc_mesh = plsc.VectorSubcoreMesh(core_axis_name="c", subcore_axis_name="s")


@functools.partial(
    pl.kernel,
    out_type=jax.ShapeDtypeStruct((NC, N_PAD, F), jnp.float32),
    mesh=_sc_mesh,
    scratch_types=[
        [pltpu.VMEM((B,), jnp.int32)] * IDEPTH,
        [pltpu.VMEM((B,), jnp.int32)] * IDEPTH,
        [pltpu.VMEM((B, F), jnp.float32)] * NBUF,
        [pltpu.VMEM((B, F), jnp.float32)] * NBUF,
        pltpu.VMEM_SHARED((N_PAD, F), jnp.float32),
        pltpu.SemaphoreType.DMA((IDEPTH,)),
        pltpu.SemaphoreType.DMA((IDEPTH,)),
        pltpu.SemaphoreType.DMA((NBUF,)),
        pltpu.SemaphoreType.DMA((NBUF,)),
        pltpu.SemaphoreType.DMA((NBUF,)),
    ],
)
def _sc_agg(emb_hbm, idxj_hbm, idxi_hbm, w_hbm, zeros_hbm, out_hbm,
            idxj_r, idxi_r, rows, wv, acc_sh,
            ij_sem, ii_sem, g_sem, w_sem, s_sem):
    cid = lax.axis_index("c")
    sid = lax.axis_index("s")
    wid = sid * NC + cid
    r0 = sid * ROWS_PER_TILE
    base0 = wid * E_PER_W

    def start_idx(kk, i6):
        pltpu.async_copy(idxj_hbm.at[pl.ds(base0 + kk * B, B)],
                         idxj_r[i6], ij_sem.at[i6])
        pltpu.async_copy(idxi_hbm.at[pl.ds(base0 + kk * B, B)],
                         idxi_r[i6], ii_sem.at[i6])

    def wait_idxj(i6):
        pltpu.make_async_copy(idxj_hbm.at[pl.ds(0, B)],
                              idxj_r[i6], ij_sem.at[i6]).wait()

    def wait_idxi(i6):
        pltpu.make_async_copy(idxi_hbm.at[pl.ds(0, B)],
                              idxi_r[i6], ii_sem.at[i6]).wait()

    def start_data(kk, i3, i6):
        pltpu.async_copy(emb_hbm.at[idxj_r[i6]], rows[i3], g_sem.at[i3])
        pltpu.async_copy(w_hbm.at[pl.ds(base0 + kk * B, B)], wv[i3],
                         w_sem.at[i3])

    def wait_scatter(i3):
        pltpu.make_async_copy(rows[i3], acc_sh.at[idxi_r[0]],
                              s_sem.at[i3]).wait()

    def process(kq, q, guard):
        i3, i6 = q % NBUF, q % IDEPTH
        n3, n6 = (q + 1) % NBUF, (q + 1) % IDEPTH

        if guard is None or kq + 2 < NBLK:
            start_idx(kq + 2, (q + 2) % IDEPTH)
        if guard is None or kq + 1 < NBLK:
            if guard is None:
                @pl.when(kq >= NBUF - 1)
                def _():
                    wait_scatter(n3)
            elif kq + 1 >= NBUF:
                wait_scatter(n3)
            wait_idxj(n6)
            start_data(kq + 1, n3, n6)

        pltpu.make_async_copy(emb_hbm.at[idxj_r[i6]], rows[i3],
                              g_sem.at[i3]).wait()
        pltpu.make_async_copy(w_hbm.at[pl.ds(0, B)], wv[i3],
                              w_sem.at[i3]).wait()

        @plsc.parallel_loop(0, B, step=1, unroll=4)
        def _p(p):
            for c in range(0, F, 16):
                slc = (pl.ds(p, 1), pl.ds(c, 16))
                rows[i3].at[slc][...] = (rows[i3].at[slc][...]
                                         * wv[i3].at[slc][...])

        wait_idxi(i6)
        pltpu.async_copy(rows[i3], acc_sh.at[idxi_r[i6]],
                         s_sem.at[i3], add=True)

    start_idx(0, 0)
    start_idx(1, 1)
    wait_idxj(0)
    start_data(0, 0, 0)
    pltpu.sync_copy(zeros_hbm.at[pl.ds(r0, ROWS_PER_TILE)],
                    acc_sh.at[pl.ds(r0, ROWS_PER_TILE)])
    plsc.subcore_barrier()

    @pl.loop(0, NBLK - 4, step=IDEPTH)
    def _blk(kk):
        for q in range(IDEPTH):
            process(kk + q, q, None)

    for kq in range(NBLK - 4, NBLK):
        process(kq, kq % IDEPTH, "tail")

    for kq in range(NBLK - NBUF, NBLK):
        wait_scatter(kq % NBUF)

    plsc.subcore_barrier()
    pltpu.sync_copy(acc_sh.at[pl.ds(r0, ROWS_PER_TILE)],
                    out_hbm.at[cid, pl.ds(r0, ROWS_PER_TILE)])


def kernel(atomic_embedding, pair_indices, f_ij, f_ij_cutoff,
           W_in, Wf1, bf1, Wf2, bf2, W2, b2, W3, b3):
    emb = pl.pallas_call(
        _emb_body,
        out_shape=jax.ShapeDtypeStruct((N, F), jnp.float32),
    )(atomic_embedding, W_in)

    w_ij = _filter_net(f_ij, f_ij_cutoff, Wf1, bf1.reshape(1, F),
                       Wf2, bf2.reshape(1, F))

    idx_i = pair_indices[0]
    idx_j = pair_indices[1]
    zeros = jnp.zeros((N_PAD, F), jnp.float32)

    partials = _sc_agg(emb, idx_j, idx_i, w_ij, zeros)

    out = pl.pallas_call(
        _out_body,
        out_shape=jax.ShapeDtypeStruct((N, F), jnp.float32),
    )(partials, W2, b2.reshape(1, F), W3, b3.reshape(1, F))
    return out


# speedup vs baseline: 3.7392x; 1.0101x over previous
import functools

import jax
import jax.numpy as jnp
from jax import lax
from jax.experimental import pallas as pl
from jax.experimental.pallas import tpu as pltpu
from jax.experimental.pallas import tpu_sc as plsc

N = 10000
E = 320000
F = 128
RBF = 20

NC = 2
NS = 16
NW = NC * NS
E_PER_W = E // NW
B = 40
NBLK = E_PER_W // B
N_PAD = 10240
ROWS_PER_TILE = N_PAD // NS

NBUF = 3
IDEPTH = 6

_LOG2 = 0.6931471805599453


def _ssp(x):
    return jnp.logaddexp(x, 0.0) - _LOG2


def _emb_body(x_ref, w_ref, o_ref):
    o_ref[...] = jnp.dot(x_ref[...], w_ref[...],
                         preferred_element_type=jnp.float32)


def _filter_body(f_ref, c_ref, wf1_ref, bf1_ref, wf2_ref, bf2_ref, o_ref):
    x = (jnp.dot(f_ref[...].astype(jnp.bfloat16),
                 wf1_ref[...].astype(jnp.bfloat16),
                 preferred_element_type=jnp.float32)
         + bf1_ref[...]).astype(jnp.bfloat16)
    h = (jnp.maximum(x, 0)
         + jnp.log1p(jnp.exp(-jnp.abs(x)))
         - jnp.bfloat16(_LOG2))
    w = jnp.dot(h, wf2_ref[...].astype(jnp.bfloat16),
                preferred_element_type=jnp.float32) + bf2_ref[...]
    o_ref[...] = w * c_ref[...]


def _out_body(p_ref, w2_ref, b2_ref, w3_ref, b3_ref, o_ref):
    agg = p_ref[0, :N, :] + p_ref[1, :N, :]
    h = _ssp(jnp.dot(agg, w2_ref[...],
                     preferred_element_type=jnp.float32) + b2_ref[...])
    o_ref[...] = jnp.dot(h, w3_ref[...],
                         preferred_element_type=jnp.float32) + b3_ref[...]


_BE = 8000


def _filter_net(f_ij, f_ij_cutoff, Wf1, bf1, Wf2, bf2):
    grid = (E // _BE,)
    return pl.pallas_call(
        _filter_body,
        grid=grid,
        in_specs=[
            pl.BlockSpec((_BE, RBF), lambda i: (i, 0)),
            pl.BlockSpec((_BE, 1), lambda i: (i, 0)),
            pl.BlockSpec((RBF, F), lambda i: (0, 0)),
            pl.BlockSpec((1, F), lambda i: (0, 0)),
            pl.BlockSpec((F, F), lambda i: (0, 0)),
            pl.BlockSpec((1, F), lambda i: (0, 0)),
        ],
        out_specs=pl.BlockSpec((_BE, F), lambda i: (i, 0)),
        out_shape=jax.ShapeDtypeStruct((E, F), jnp.float32),
    )(f_ij, f_ij_cutoff, Wf1, bf1, Wf2, bf2)


_sc_mesh = plsc.VectorSubcoreMesh(core_axis_name="c", subcore_axis_name="s")


@functools.partial(
    pl.kernel,
    out_type=jax.ShapeDtypeStruct((NC, N_PAD, F), jnp.float32),
    mesh=_sc_mesh,
    scratch_types=[
        [pltpu.VMEM((B,), jnp.int32)] * IDEPTH,
        [pltpu.VMEM((B,), jnp.int32)] * IDEPTH,
        [pltpu.VMEM((B, F), jnp.float32)] * NBUF,
        [pltpu.VMEM((B, F), jnp.float32)] * NBUF,
        pltpu.VMEM_SHARED((N_PAD, F), jnp.float32),
        pltpu.SemaphoreType.DMA((IDEPTH,)),
        pltpu.SemaphoreType.DMA((IDEPTH,)),
        pltpu.SemaphoreType.DMA((NBUF,)),
        pltpu.SemaphoreType.DMA((NBUF,)),
        pltpu.SemaphoreType.DMA((NBUF,)),
    ],
)
def _sc_agg(emb_hbm, idxj_hbm, idxi_hbm, w_hbm, zeros_hbm, out_hbm,
            idxj_r, idxi_r, rows, wv, acc_sh,
            ij_sem, ii_sem, g_sem, w_sem, s_sem):
    cid = lax.axis_index("c")
    sid = lax.axis_index("s")
    wid = sid * NC + cid
    r0 = sid * ROWS_PER_TILE
    base0 = wid * E_PER_W

    def start_idx(kk, i6):
        pltpu.async_copy(idxj_hbm.at[pl.ds(base0 + kk * B, B)],
                         idxj_r[i6], ij_sem.at[i6])
        pltpu.async_copy(idxi_hbm.at[pl.ds(base0 + kk * B, B)],
                         idxi_r[i6], ii_sem.at[i6])

    def wait_idxj(i6):
        pltpu.make_async_copy(idxj_hbm.at[pl.ds(0, B)],
                              idxj_r[i6], ij_sem.at[i6]).wait()

    def wait_idxi(i6):
        pltpu.make_async_copy(idxi_hbm.at[pl.ds(0, B)],
                              idxi_r[i6], ii_sem.at[i6]).wait()

    def start_data(kk, i3, i6):
        pltpu.async_copy(emb_hbm.at[idxj_r[i6]], rows[i3], g_sem.at[i3])
        pltpu.async_copy(w_hbm.at[pl.ds(base0 + kk * B, B)], wv[i3],
                         w_sem.at[i3])

    def wait_scatter(i3):
        pltpu.make_async_copy(rows[i3], acc_sh.at[idxi_r[0]],
                              s_sem.at[i3]).wait()

    def process(kq, q, guard):
        i3, i6 = q % NBUF, q % IDEPTH
        n3, n6 = (q + 1) % NBUF, (q + 1) % IDEPTH

        if guard is None or kq + 2 < NBLK:
            start_idx(kq + 2, (q + 2) % IDEPTH)
        if guard is None or kq + 1 < NBLK:
            if guard is None:
                @pl.when(kq >= NBUF - 1)
                def _():
                    wait_scatter(n3)
            elif kq + 1 >= NBUF:
                wait_scatter(n3)
            wait_idxj(n6)
            start_data(kq + 1, n3, n6)

        pltpu.make_async_copy(emb_hbm.at[idxj_r[i6]], rows[i3],
                              g_sem.at[i3]).wait()
        pltpu.make_async_copy(w_hbm.at[pl.ds(0, B)], wv[i3],
                              w_sem.at[i3]).wait()

        @plsc.parallel_loop(0, B, step=1, unroll=4)
        def _p(p):
            for c in range(0, F, 16):
                slc = (pl.ds(p, 1), pl.ds(c, 16))
                rows[i3].at[slc][...] = (rows[i3].at[slc][...]
                                         * wv[i3].at[slc][...])

        wait_idxi(i6)
        pltpu.async_copy(rows[i3], acc_sh.at[idxi_r[i6]],
                         s_sem.at[i3], add=True)

    start_idx(0, 0)
    start_idx(1, 1)
    wait_idxj(0)
    start_data(0, 0, 0)
    pltpu.sync_copy(zeros_hbm.at[pl.ds(r0, ROWS_PER_TILE)],
                    acc_sh.at[pl.ds(r0, ROWS_PER_TILE)])
    plsc.subcore_barrier()

    @pl.loop(0, NBLK - 4, step=IDEPTH)
    def _blk(kk):
        for q in range(IDEPTH):
            process(kk + q, q, None)

    for kq in range(NBLK - 4, NBLK):
        process(kq, kq % IDEPTH, "tail")

    for kq in range(NBLK - NBUF, NBLK):
        wait_scatter(kq % NBUF)

    plsc.subcore_barrier()
    pltpu.sync_copy(acc_sh.at[pl.ds(r0, ROWS_PER_TILE)],
                    out_hbm.at[cid, pl.ds(r0, ROWS_PER_TILE)])


def kernel(atomic_embedding, pair_indices, f_ij, f_ij_cutoff,
           W_in, Wf1, bf1, Wf2, bf2, W2, b2, W3, b3):
    emb = pl.pallas_call(
        _emb_body,
        out_shape=jax.ShapeDtypeStruct((N, F), jnp.float32),
    )(atomic_embedding, W_in)

    w_ij = _filter_net(f_ij, f_ij_cutoff, Wf1, bf1.reshape(1, F),
                       Wf2, bf2.reshape(1, F))

    idx_i = pair_indices[0]
    idx_j = pair_indices[1]
    zeros = jnp.zeros((N_PAD, F), jnp.float32)

    partials = _sc_agg(emb, idx_j, idx_i, w_ij, zeros)

    out = pl.pallas_call(
        _out_body,
        out_shape=jax.ShapeDtypeStruct((N, F), jnp.float32),
    )(partials, W2, b2.reshape(1, F), W3, b3.reshape(1, F))
    return out
